# Initial kernel scaffold; baseline (speedup 1.0000x reference)
import jax
import jax.numpy as jnp
from jax.experimental import pallas as pl


def kernel(x, edge_index, W1, b1, W2, b2, W3, b3, Wp, bp, Wc, bc):
    raise NotImplementedError("write your pallas kernel here")


# speedup vs baseline: 16.6910x; 16.6910x over previous
import functools

import jax
import jax.numpy as jnp
from jax import lax
from jax.experimental import pallas as pl
from jax.experimental.pallas import tpu as pltpu
from jax.experimental.pallas import tpu_sc as plsc

N = 10000
E = 320000
D = 128

NC, NS, L = 2, 16, 16
NW = NC * NS
EPW = E // NW
C = 80
NCHUNK = EPW // C
NP = 10240
RPT = NP // NS

_mesh = plsc.VectorSubcoreMesh(core_axis_name="c", subcore_axis_name="s")
_params = pltpu.CompilerParams(needs_layout_passes=False)


@functools.partial(
    pl.kernel,
    out_type=jax.ShapeDtypeStruct((NW, 1, N), jnp.float32),
    mesh=_mesh,
    compiler_params=_params,
    scratch_types=[
        pltpu.VMEM((C,), jnp.int32),
        pltpu.VMEM((1, N), jnp.float32),
    ],
)
def _deg_kernel(dst_hbm, out_hbm, idx_d, acc):
    cid = lax.axis_index("c")
    sid = lax.axis_index("s")
    wid = sid * NC + cid
    zf = jnp.zeros((L,), jnp.float32)

    def zinit(i, _):
        acc[0, pl.ds(i * L, L)] = zf
        return 0

    lax.fori_loop(0, N // L, zinit, 0)

    row0 = jnp.zeros((L,), jnp.int32)
    onesv = jnp.ones((L,), jnp.float32)

    def body(g, _):
        base = wid * EPW + g * C
        pltpu.sync_copy(dst_hbm.at[pl.ds(base, C)], idx_d)
        for k in range(C // L):
            iv = idx_d[pl.ds(k * L, L)]
            plsc.addupdate_scatter(acc, [row0, iv], onesv)
        return 0

    lax.fori_loop(0, NCHUNK, body, 0)
    pltpu.sync_copy(acc, out_hbm.at[wid])


@functools.partial(
    pl.kernel,
    out_type=(jax.ShapeDtypeStruct((NC, NP, D), jnp.float32),
              jax.ShapeDtypeStruct((NW, 1, N), jnp.float32)),
    mesh=_mesh,
    compiler_params=_params,
    scratch_types=[
        pltpu.VMEM((C,), jnp.int32),
        pltpu.VMEM((C,), jnp.int32),
        pltpu.VMEM((C, D), jnp.float32),
        pltpu.VMEM((N,), jnp.float32),
        pltpu.VMEM((1, N), jnp.float32),
        pltpu.VMEM_SHARED((NP, D), jnp.float32),
        pltpu.SemaphoreType.DMA,
    ],
)
def _agg_w_kernel(z_hbm, src_hbm, dst_hbm, dinv_hbm, zeros_hbm,
                  out_hbm, wout_hbm, idx_s, idx_d, rows, dinv_v, wacc,
                  acc, sem):
    cid = lax.axis_index("c")
    sid = lax.axis_index("s")
    wid = sid * NC + cid
    sl = pl.ds(sid * RPT, RPT)
    @pl.when(cid == 0)
    def _():
        pltpu.sync_copy(z_hbm.at[sl], acc.at[sl])

    @pl.when(cid != 0)
    def _():
        pltpu.sync_copy(zeros_hbm.at[sl], acc.at[sl])

    pltpu.sync_copy(dinv_hbm, dinv_v)
    zf = jnp.zeros((L,), jnp.float32)

    def zinit(i, _):
        wacc[0, pl.ds(i * L, L)] = zf
        return 0

    lax.fori_loop(0, N // L, zinit, 0)
    plsc.subcore_barrier()

    row0 = jnp.zeros((L,), jnp.int32)

    def body(g, _):
        base = wid * EPW + g * C
        pltpu.sync_copy(src_hbm.at[pl.ds(base, C)], idx_s)
        pltpu.sync_copy(dst_hbm.at[pl.ds(base, C)], idx_d)
        pltpu.async_copy(z_hbm.at[idx_s], rows, sem).wait()
        pltpu.sync_copy(rows, acc.at[idx_d], add=True)
        for k in range(C // L):
            iv_d = idx_d[pl.ds(k * L, L)]
            iv_s = idx_s[pl.ds(k * L, L)]
            dv = plsc.load_gather(dinv_v, [iv_d])
            plsc.addupdate_scatter(wacc, [row0, iv_s], dv)
        return 0

    lax.fori_loop(0, NCHUNK, body, 0)
    plsc.subcore_barrier()
    pltpu.sync_copy(acc.at[sl], out_hbm.at[cid, sl])
    pltpu.sync_copy(wacc, wout_hbm.at[wid])


@functools.partial(
    pl.kernel,
    out_type=jax.ShapeDtypeStruct((NC, NP, D), jnp.float32),
    mesh=_mesh,
    compiler_params=_params,
    scratch_types=[
        pltpu.VMEM((C,), jnp.int32),
        pltpu.VMEM((C,), jnp.int32),
        pltpu.VMEM((C, D), jnp.float32),
        pltpu.VMEM_SHARED((NP, D), jnp.float32),
        pltpu.SemaphoreType.DMA,
    ],
)
def _agg_kernel(z_hbm, src_hbm, dst_hbm, zeros_hbm,
                out_hbm, idx_s, idx_d, rows, acc, sem):
    cid = lax.axis_index("c")
    sid = lax.axis_index("s")
    wid = sid * NC + cid
    sl = pl.ds(sid * RPT, RPT)
    @pl.when(cid == 0)
    def _():
        pltpu.sync_copy(z_hbm.at[sl], acc.at[sl])

    @pl.when(cid != 0)
    def _():
        pltpu.sync_copy(zeros_hbm.at[sl], acc.at[sl])

    plsc.subcore_barrier()

    def body(g, _):
        base = wid * EPW + g * C
        pltpu.sync_copy(src_hbm.at[pl.ds(base, C)], idx_s)
        pltpu.sync_copy(dst_hbm.at[pl.ds(base, C)], idx_d)
        pltpu.async_copy(z_hbm.at[idx_s], rows, sem).wait()
        pltpu.sync_copy(rows, acc.at[idx_d], add=True)
        return 0

    lax.fori_loop(0, NCHUNK, body, 0)
    plsc.subcore_barrier()
    pltpu.sync_copy(acc.at[sl], out_hbm.at[cid, sl])


def kernel(x, edge_index, W1, b1, W2, b2, W3, b3, Wp, bp, Wc, bc):
    src = edge_index[0]
    dst = edge_index[1]
    zerosd = jnp.zeros((NP, D), jnp.float32)
    pad = ((0, NP - N), (0, 0))

    degp = _deg_kernel(dst)
    deg = jnp.sum(degp[:, 0, :], axis=0) + 1.0
    dinv = lax.rsqrt(deg)

    z1 = jnp.pad((x @ W1) * dinv[:, None], pad)
    p, wp_ = _agg_w_kernel(z1, src, dst, dinv, zerosd)
    h1 = jax.nn.relu((p[0, :N] + p[1, :N]) * dinv[:, None] + b1)

    z2 = jnp.pad((h1 @ W2) * dinv[:, None], pad)
    q = _agg_kernel(z2, src, dst, zerosd)
    h2 = jax.nn.relu((q[0, :N] + q[1, :N]) * dinv[:, None] + b2)

    w = jnp.sum(wp_[:, 0, :], axis=0)
    u = dinv * (dinv + w)
    t = u[None, :] @ h2
    g = (t @ W3) * (1.0 / N) + b3
    g = jax.nn.relu(g @ Wp + bp)
    return g @ Wc + bc
